# Initial kernel scaffold; baseline (speedup 1.0000x reference)
import jax
import jax.numpy as jnp
from jax.experimental import pallas as pl


def kernel(inputs, embeddings):
    raise NotImplementedError("write your pallas kernel here")


# speedup vs baseline: 1.1248x; 1.1248x over previous
import functools

import jax
import jax.numpy as jnp
from jax import lax
from jax.experimental import pallas as pl
from jax.experimental.pallas import tpu as pltpu
from jax.experimental.pallas import tpu_sc as plsc

NUM_EMB = 8192
DIM = 256
N_ROWS = 16 * 1024

R_BLK = 256
I_BLKS = N_ROWS // R_BLK
CHUNKS = ((0, 2816), (2816, 5632), (5632, 8192))


def _argmin_body(x_ref, e_ref, idx_ref, et_ref):
    i = pl.program_id(0)
    x = x_ref[...]
    xb = x.astype(jnp.bfloat16)
    nx = jnp.sum(x * x, axis=1, keepdims=True)
    big = jnp.int32(2**30)
    bv16 = None
    bi = None
    for (lo, hi) in CHUNKS:
        w = hi - lo
        e = e_ref[:, pl.ds(lo, w)]
        eb = e.astype(jnp.bfloat16)
        p = jnp.dot(xb, eb, preferred_element_type=jnp.float32)
        ne = jnp.sum(e * e, axis=0, keepdims=True)
        d = (nx - 2.0 * p) + ne
        m = jnp.min(d, axis=1, keepdims=True)
        iota = lax.broadcasted_iota(jnp.int32, (R_BLK, w), 1)
        bidx = jnp.min(jnp.where(d == m, iota, big), axis=1, keepdims=True)
        gidx = bidx + lo
        m16 = m.astype(jnp.bfloat16).astype(jnp.float32)
        if bv16 is None:
            bv16 = m16
            bi = gidx
        else:
            win = m < bv16
            bv16 = jnp.where(win, m16, bv16)
            bi = jnp.where(win, gidx, bi)
    idx_ref[...] = bi

    @pl.when(i == 0)
    def _():
        et_ref[...] = e_ref[...].T


def _argmin_et_call(flat_x, emb):
    return pl.pallas_call(
        _argmin_body,
        grid=(I_BLKS,),
        in_specs=[
            pl.BlockSpec((R_BLK, DIM), lambda i: (i, 0)),
            pl.BlockSpec((DIM, NUM_EMB), lambda i: (0, 0)),
        ],
        out_specs=[
            pl.BlockSpec((R_BLK, 1), lambda i: (i, 0)),
            pl.BlockSpec((NUM_EMB, DIM), lambda i: (0, 0)),
        ],
        out_shape=[
            jax.ShapeDtypeStruct((N_ROWS, 1), jnp.int32),
            jax.ShapeDtypeStruct((NUM_EMB, DIM), jnp.float32),
        ],
    )(flat_x, emb)


_SC_CHUNK = 128


def _make_sc_gather():
    info = plsc.get_sparse_core_info()
    nc, ns = info.num_cores, info.num_subcores
    nw = nc * ns
    rows_per_w = N_ROWS // nw
    n_chunks = rows_per_w // _SC_CHUNK
    mesh = plsc.VectorSubcoreMesh(core_axis_name="c", subcore_axis_name="s")

    @functools.partial(
        pl.kernel,
        mesh=mesh,
        out_type=[
            jax.ShapeDtypeStruct((N_ROWS, DIM), jnp.float32),
            jax.ShapeDtypeStruct((nw, 16), jnp.float32),
        ],
        scratch_types=[
            pltpu.VMEM((_SC_CHUNK,), jnp.int32),
            pltpu.VMEM((_SC_CHUNK, DIM), jnp.float32),
            pltpu.VMEM((_SC_CHUNK, DIM), jnp.float32),
            pltpu.VMEM((16,), jnp.float32),
            pltpu.SemaphoreType.DMA,
        ],
    )
    def sc_fn(et_hbm, idx_hbm, x_hbm, out_hbm, loss_hbm, idx_v, rows_v, x_v,
              acc_v, sem):
        wid = lax.axis_index("s") * nc + lax.axis_index("c")
        base = wid * rows_per_w
        acc = jnp.zeros((16,), jnp.float32)
        for c in range(n_chunks):
            rbase = base + c * _SC_CHUNK
            pltpu.sync_copy(idx_hbm.at[pl.ds(rbase, _SC_CHUNK)], idx_v)
            pltpu.async_copy(et_hbm.at[idx_v], rows_v, sem).wait()
            pltpu.sync_copy(x_hbm.at[pl.ds(rbase, _SC_CHUNK), :], x_v)

            def row_body(r, a):
                for v in range(DIM // 16):
                    sl = pl.ds(v * 16, 16)
                    q = rows_v[r, sl]
                    xv = x_v[r, sl]
                    dlt = q - xv
                    rows_v[r, sl] = xv + dlt
                    a = a + dlt * dlt
                return a

            acc = lax.fori_loop(0, _SC_CHUNK, row_body, acc)
            pltpu.sync_copy(rows_v, out_hbm.at[pl.ds(rbase, _SC_CHUNK), :])
        acc_v[...] = acc
        pltpu.sync_copy(acc_v, loss_hbm.at[wid])

    return sc_fn


_sc_gather_call = None


def kernel(inputs, embeddings):
    global _sc_gather_call
    if _sc_gather_call is None:
        _sc_gather_call = _make_sc_gather()
    input_shape = inputs.shape
    flat_x = inputs.reshape(-1, DIM)
    idx2d, et = _argmin_et_call(flat_x, embeddings)
    q_st, loss_parts = _sc_gather_call(et, idx2d.reshape(-1), flat_x)
    commitment_loss = 0.25 * (jnp.sum(loss_parts) / float(N_ROWS * DIM))
    return (q_st.reshape(input_shape), commitment_loss, idx2d.reshape(-1))


# speedup vs baseline: 1.2580x; 1.1184x over previous
import functools

import jax
import jax.numpy as jnp
from jax import lax
from jax.experimental import pallas as pl
from jax.experimental.pallas import tpu as pltpu
from jax.experimental.pallas import tpu_sc as plsc

NUM_EMB = 8192
DIM = 256
N_ROWS = 16 * 1024

R_BLK = 256
I_BLKS = N_ROWS // R_BLK
CHUNKS = ((0, 2816), (2816, 5632), (5632, 8192))


def _argmin_body(x_ref, e_ref, eb_ref, nx_ref, ne_ref, idx_ref, et_ref):
    i = pl.program_id(0)

    @pl.when(i == 0)
    def _():
        et_ref[...] = e_ref[...].T

    x = x_ref[...]
    xb = x.astype(jnp.bfloat16)
    nx = nx_ref[...]
    big = jnp.int32(2**30)
    bv16 = None
    bi = None
    for (lo, hi) in CHUNKS:
        w = hi - lo
        eb = eb_ref[:, pl.ds(lo, w)]
        p = jnp.dot(xb, eb, preferred_element_type=jnp.float32)
        ne = ne_ref[:, pl.ds(lo, w)]
        d = (nx - 2.0 * p) + ne
        m = jnp.min(d, axis=1, keepdims=True)
        iota = lax.broadcasted_iota(jnp.int32, (1, w), 1)
        bidx = jnp.min(jnp.where(d == m, iota, big), axis=1, keepdims=True)
        gidx = bidx + lo
        m16 = m.astype(jnp.bfloat16).astype(jnp.float32)
        if bv16 is None:
            bv16 = m16
            bi = gidx
        else:
            win = m < bv16
            bv16 = jnp.where(win, m16, bv16)
            bi = jnp.where(win, gidx, bi)
    idx_ref[...] = bi


def _argmin_et_call(flat_x, emb):
    nx = jnp.sum(flat_x ** 2, axis=1, keepdims=True)
    ne = jnp.sum(emb ** 2, axis=0, keepdims=True)
    return pl.pallas_call(
        _argmin_body,
        grid=(I_BLKS,),
        in_specs=[
            pl.BlockSpec((R_BLK, DIM), lambda i: (i, 0)),
            pl.BlockSpec((DIM, NUM_EMB), lambda i: (0, 0)),
            pl.BlockSpec((DIM, NUM_EMB), lambda i: (0, 0)),
            pl.BlockSpec((R_BLK, 1), lambda i: (i, 0)),
            pl.BlockSpec((1, NUM_EMB), lambda i: (0, 0)),
        ],
        out_specs=[
            pl.BlockSpec((R_BLK, 1), lambda i: (i, 0)),
            pl.BlockSpec((NUM_EMB, DIM), lambda i: (0, 0)),
        ],
        out_shape=[
            jax.ShapeDtypeStruct((N_ROWS, 1), jnp.int32),
            jax.ShapeDtypeStruct((NUM_EMB, DIM), jnp.float32),
        ],
    )(flat_x, emb, emb.astype(jnp.bfloat16), nx, ne)


_SC_CHUNK = 128


def _make_sc_gather():
    info = plsc.get_sparse_core_info()
    nc, ns = info.num_cores, info.num_subcores
    nw = nc * ns
    rows_per_w = N_ROWS // nw
    n_chunks = rows_per_w // _SC_CHUNK
    mesh = plsc.VectorSubcoreMesh(core_axis_name="c", subcore_axis_name="s")

    @functools.partial(
        pl.kernel,
        mesh=mesh,
        out_type=[
            jax.ShapeDtypeStruct((N_ROWS, DIM), jnp.float32),
            jax.ShapeDtypeStruct((nw, 16), jnp.float32),
        ],
        scratch_types=[
            pltpu.VMEM((_SC_CHUNK,), jnp.int32),
            pltpu.VMEM((_SC_CHUNK, DIM), jnp.float32),
            pltpu.VMEM((_SC_CHUNK, DIM), jnp.float32),
            pltpu.VMEM((16,), jnp.float32),
            pltpu.SemaphoreType.DMA,
        ],
    )
    def sc_fn(et_hbm, idx_hbm, x_hbm, out_hbm, loss_hbm, idx_v, rows_v, x_v,
              acc_v, sem):
        wid = lax.axis_index("s") * nc + lax.axis_index("c")
        base = wid * rows_per_w
        acc = jnp.zeros((16,), jnp.float32)
        for c in range(n_chunks):
            rbase = base + c * _SC_CHUNK
            pltpu.sync_copy(idx_hbm.at[pl.ds(rbase, _SC_CHUNK)], idx_v)
            pltpu.async_copy(et_hbm.at[idx_v], rows_v, sem).wait()
            pltpu.sync_copy(x_hbm.at[pl.ds(rbase, _SC_CHUNK), :], x_v)

            def row_body(r, a):
                for v in range(DIM // 16):
                    sl = pl.ds(v * 16, 16)
                    q = rows_v[r, sl]
                    xv = x_v[r, sl]
                    dlt = q - xv
                    rows_v[r, sl] = xv + dlt
                    a = a + dlt * dlt
                return a

            acc = lax.fori_loop(0, _SC_CHUNK, row_body, acc)
            pltpu.sync_copy(rows_v, out_hbm.at[pl.ds(rbase, _SC_CHUNK), :])
        acc_v[...] = acc
        pltpu.sync_copy(acc_v, loss_hbm.at[wid])

    return sc_fn


_sc_gather_call = None


def kernel(inputs, embeddings):
    global _sc_gather_call
    if _sc_gather_call is None:
        _sc_gather_call = _make_sc_gather()
    input_shape = inputs.shape
    flat_x = inputs.reshape(-1, DIM)
    idx2d, et = _argmin_et_call(flat_x, embeddings)
    q_st, loss_parts = _sc_gather_call(et, idx2d.reshape(-1), flat_x)
    commitment_loss = 0.25 * (jnp.sum(loss_parts) / float(N_ROWS * DIM))
    return (q_st.reshape(input_shape), commitment_loss, idx2d.reshape(-1))


# speedup vs baseline: 1.3377x; 1.0633x over previous
import functools

import jax
import jax.numpy as jnp
from jax import lax
from jax.experimental import pallas as pl
from jax.experimental.pallas import tpu as pltpu
from jax.experimental.pallas import tpu_sc as plsc

NUM_EMB = 8192
DIM = 256
N_ROWS = 16 * 1024

R_BLK = 512
I_BLKS = N_ROWS // R_BLK
CHUNKS = ((0, 2816), (2816, 5632), (5632, 8192))


def _argmin_body(x_ref, e_ref, eb_ref, nx_ref, ne_ref, idx_ref, et_ref):
    i = pl.program_id(0)

    @pl.when(i == 0)
    def _():
        et_ref[...] = e_ref[...].T

    x = x_ref[...]
    xb = x.astype(jnp.bfloat16)
    nx = nx_ref[...]
    big = jnp.int32(2**30)
    bv16 = None
    bi = None
    for (lo, hi) in CHUNKS:
        w = hi - lo
        eb = eb_ref[:, pl.ds(lo, w)]
        p = jnp.dot(xb, eb, preferred_element_type=jnp.float32)
        ne = ne_ref[:, pl.ds(lo, w)]
        d = (nx - 2.0 * p) + ne
        m = jnp.min(d, axis=1, keepdims=True)
        iota = lax.broadcasted_iota(jnp.int32, (1, w), 1)
        bidx = jnp.min(jnp.where(d == m, iota, big), axis=1, keepdims=True)
        gidx = bidx + lo
        m16 = m.astype(jnp.bfloat16).astype(jnp.float32)
        if bv16 is None:
            bv16 = m16
            bi = gidx
        else:
            win = m < bv16
            bv16 = jnp.where(win, m16, bv16)
            bi = jnp.where(win, gidx, bi)
    idx_ref[...] = bi


def _argmin_et_call(flat_x, emb):
    nx = jnp.sum(flat_x ** 2, axis=1, keepdims=True)
    ne = jnp.sum(emb ** 2, axis=0, keepdims=True)
    return pl.pallas_call(
        _argmin_body,
        grid=(I_BLKS,),
        in_specs=[
            pl.BlockSpec((R_BLK, DIM), lambda i: (i, 0)),
            pl.BlockSpec((DIM, NUM_EMB), lambda i: (0, 0)),
            pl.BlockSpec((DIM, NUM_EMB), lambda i: (0, 0)),
            pl.BlockSpec((R_BLK, 1), lambda i: (i, 0)),
            pl.BlockSpec((1, NUM_EMB), lambda i: (0, 0)),
        ],
        out_specs=[
            pl.BlockSpec((R_BLK, 1), lambda i: (i, 0)),
            pl.BlockSpec((NUM_EMB, DIM), lambda i: (0, 0)),
        ],
        out_shape=[
            jax.ShapeDtypeStruct((N_ROWS, 1), jnp.int32),
            jax.ShapeDtypeStruct((NUM_EMB, DIM), jnp.float32),
        ],
    )(flat_x, emb, emb.astype(jnp.bfloat16), nx, ne)


_SC_CHUNK = 128


def _make_sc_gather():
    info = plsc.get_sparse_core_info()
    nc, ns = info.num_cores, info.num_subcores
    nw = nc * ns
    rows_per_w = N_ROWS // nw
    n_chunks = rows_per_w // _SC_CHUNK
    mesh = plsc.VectorSubcoreMesh(core_axis_name="c", subcore_axis_name="s")

    @functools.partial(
        pl.kernel,
        mesh=mesh,
        out_type=[
            jax.ShapeDtypeStruct((N_ROWS, DIM), jnp.float32),
            jax.ShapeDtypeStruct((nw, 16), jnp.float32),
        ],
        scratch_types=[
            pltpu.VMEM((_SC_CHUNK,), jnp.int32),
            pltpu.VMEM((_SC_CHUNK, DIM), jnp.float32),
            pltpu.VMEM((_SC_CHUNK, DIM), jnp.float32),
            pltpu.VMEM((16,), jnp.float32),
            pltpu.SemaphoreType.DMA,
        ],
    )
    def sc_fn(et_hbm, idx_hbm, x_hbm, out_hbm, loss_hbm, idx_v, rows_v, x_v,
              acc_v, sem):
        wid = lax.axis_index("s") * nc + lax.axis_index("c")
        base = wid * rows_per_w
        acc = jnp.zeros((16,), jnp.float32)
        for c in range(n_chunks):
            rbase = base + c * _SC_CHUNK
            pltpu.sync_copy(idx_hbm.at[pl.ds(rbase, _SC_CHUNK)], idx_v)
            pltpu.async_copy(et_hbm.at[idx_v], rows_v, sem).wait()
            pltpu.sync_copy(x_hbm.at[pl.ds(rbase, _SC_CHUNK), :], x_v)

            def row_body(r, a):
                for v in range(DIM // 16):
                    sl = pl.ds(v * 16, 16)
                    q = rows_v[r, sl]
                    xv = x_v[r, sl]
                    dlt = q - xv
                    rows_v[r, sl] = xv + dlt
                    a = a + dlt * dlt
                return a

            acc = lax.fori_loop(0, _SC_CHUNK, row_body, acc)
            pltpu.sync_copy(rows_v, out_hbm.at[pl.ds(rbase, _SC_CHUNK), :])
        acc_v[...] = acc
        pltpu.sync_copy(acc_v, loss_hbm.at[wid])

    return sc_fn


_sc_gather_call = None


def kernel(inputs, embeddings):
    global _sc_gather_call
    if _sc_gather_call is None:
        _sc_gather_call = _make_sc_gather()
    input_shape = inputs.shape
    flat_x = inputs.reshape(-1, DIM)
    idx2d, et = _argmin_et_call(flat_x, embeddings)
    q_st, loss_parts = _sc_gather_call(et, idx2d.reshape(-1), flat_x)
    commitment_loss = 0.25 * (jnp.sum(loss_parts) / float(N_ROWS * DIM))
    return (q_st.reshape(input_shape), commitment_loss, idx2d.reshape(-1))
